# baseline (speedup 1.0000x reference)
import functools

import jax
import jax.numpy as jnp
from jax import lax
from jax.experimental import pallas as pl
from jax.experimental.pallas import tpu as pltpu
from jax.experimental.pallas import tpu_sc as plsc

_RADIUS = 0.083
_R2 = _RADIUS * _RADIUS
_G = 11
_NCELL = _G * _G * _G
_RSPAN = 2
_D_CAP = 96
_LANES = 16

_NW = 32

_INTERPRET = False


def _splat_i32(x):
    return jnp.zeros((_LANES,), jnp.int32) + x


def _bf16r(v):
    b = plsc.bitcast(v, jnp.int32)
    tie = lax.bitwise_and(lax.shift_right_logical(b, 16), 1)
    b = b + 0x7FFF + tie
    b = lax.bitwise_and(b, jnp.int32(-65536))
    return plsc.bitcast(b, jnp.float32)


def _sc_body(qx_h, qy_h, qz_h, px_h, py_h, pz_h, order_h, cs_h, init_h,
             table_h, edges_h, counts_h,
             qx_v, qy_v, qz_v, px_v, py_v, pz_v, order_v, cs_v, nbr_v,
             cnt_v, gbuf_v, sem, *, n_sorted, qpt, n_zero_row, csz):
    wid = lax.axis_index("s") * 2 + lax.axis_index("c")
    qbase = wid * qpt

    pltpu.sync_copy(qx_h.at[pl.ds(qbase, qpt)], qx_v)
    pltpu.sync_copy(qy_h.at[pl.ds(qbase, qpt)], qy_v)
    pltpu.sync_copy(qz_h.at[pl.ds(qbase, qpt)], qz_v)
    pltpu.sync_copy(px_h, px_v)
    pltpu.sync_copy(py_h, py_v)
    pltpu.sync_copy(pz_h, pz_v)
    pltpu.sync_copy(order_h, order_v)
    pltpu.sync_copy(cs_h, cs_v)
    pltpu.sync_copy(init_h, nbr_v)

    lane = lax.iota(jnp.int32, _LANES)
    lane0 = lane == 0

    nspan = 2 * _RSPAN + 1

    def per_query(q, _):
        qi = _splat_i32(q)
        qxv = plsc.load_gather(qx_v, [qi])
        qyv = plsc.load_gather(qy_v, [qi])
        qzv = plsc.load_gather(qz_v, [qi])
        qqv = (qxv * qxv + qyv * qyv) + qzv * qzv
        qbx = _bf16r(qxv)
        qby = _bf16r(qyv)
        qbz = _bf16r(qzv)
        cxv = jnp.clip((qxv * _G).astype(jnp.int32), 0, _G - 1)
        cyv = jnp.clip((qyv * _G).astype(jnp.int32), 0, _G - 1)
        czv = jnp.clip((qzv * _G).astype(jnp.int32), 0, _G - 1)
        zlo = jnp.maximum(czv - _RSPAN, 0)
        zhi = jnp.minimum(czv + _RSPAN, _G - 1)

        def per_run(k, cnt_vec):
            dx = k // nspan - _RSPAN
            dy = k % nspan - _RSPAN
            ax = cxv + dx
            ay = cyv + dy
            okr = (ax >= 0) & (ax < _G) & (ay >= 0) & (ay < _G)
            base = (ax * _G + ay) * _G
            lin_lo = jnp.clip(base + zlo, 0, csz - 1)
            lin_hi = jnp.clip(base + zhi + 1, 0, csz - 1)
            sv = plsc.load_gather(cs_v, [lin_lo])
            ev = plsc.load_gather(cs_v, [lin_hi])
            sv = jnp.where(okr, sv, 0)
            ev = jnp.where(okr, ev, 0)
            s_start = jnp.max(sv)
            e_end = jnp.max(ev)
            trips = (e_end - s_start + (_LANES - 1)) // _LANES

            def per_chunk(t, cnt_in):
                s0 = s_start + t * _LANES
                svec = s0 + lane
                valid = svec < e_end
                svec_c = jnp.minimum(svec, n_sorted - 1)
                ov = plsc.load_gather(order_v, [svec_c])
                xs = plsc.load_gather(px_v, [ov])
                ys = plsc.load_gather(py_v, [ov])
                zs = plsc.load_gather(pz_v, [ov])
                pp = (xs * xs + ys * ys) + zs * zs
                qp = (qbx * _bf16r(xs) + qby * _bf16r(ys)) + qbz * _bf16r(zs)
                d2 = (qqv + pp) - 2.0 * qp
                inr = valid & (d2 <= _R2)
                pcs = plsc.cumsum(jnp.where(inr, 1, 0).astype(jnp.int32))
                tgt = cnt_in + pcs - 1
                w = inr & (tgt < _D_CAP)
                flat = jnp.clip(q * _D_CAP + tgt, 0, qpt * _D_CAP - 1)
                row = lax.shift_right_logical(flat, 7)
                col = lax.bitwise_and(flat, 127)
                plsc.store_scatter(nbr_v, [row, col], ov, mask=w)
                return cnt_in + plsc.all_reduce_population_count(inr)

            return lax.fori_loop(0, trips, per_chunk, cnt_vec)

        cnt_vec = lax.fori_loop(0, nspan * nspan, per_run, _splat_i32(0))
        plsc.store_scatter(cnt_v, [qi], cnt_vec, mask=lane0)
        return _

    lax.fori_loop(0, qpt, per_query, 0)

    pltpu.sync_copy(cnt_v, counts_h.at[pl.ds(qbase, qpt)])

    nrows = qpt * _D_CAP // 128
    ebase = qbase * _D_CAP

    def per_gather(c, _):
        pltpu.async_copy(table_h.at[nbr_v.at[c]], gbuf_v, sem)
        return _

    lax.fori_loop(0, nrows, per_gather, 0)

    def per_drain(c, _):
        pltpu.make_async_copy(table_h.at[nbr_v.at[0]], gbuf_v, sem).wait()
        return _

    lax.fori_loop(0, nrows, per_drain, 0)


def _tc_body(lat_ref, edges_ref, cnt_ref,
             k0a_ref, k0b_ref, kb0_ref, k1_ref, kb1_ref, k2_ref, kb2_ref,
             p0_ref, pb0_ref, p1_ref, pb1_ref, out_ref, *, bq):
    e = edges_ref[...]
    rb = e[:, :64]
    pe = e[:, 64:67]
    lat = lat_ref[...]
    aq = jnp.dot(lat, k0a_ref[...]) + kb0_ref[...]
    aqe = jnp.broadcast_to(aq[:, None, :], (bq, _D_CAP, 64))
    aqe = aqe.reshape(bq * _D_CAP, 64)
    h1 = jax.nn.gelu(aqe + jnp.dot(pe, k0b_ref[...]))
    h2 = jax.nn.gelu(jnp.dot(h1, k1_ref[...]) + kb1_ref[...])
    kv = jnp.dot(h2, k2_ref[...]) + kb2_ref[...]
    v = kv * rb
    s = v.reshape(bq, _D_CAP, 64).sum(axis=1)
    cnt = jnp.clip(cnt_ref[...], 1.0, None)
    mean = s / cnt
    h = jax.nn.gelu(jnp.dot(mean, p0_ref[...]) + pb0_ref[...])
    out_ref[...] = jnp.dot(h, p1_ref[...]) + pb1_ref[...]


def _sc_stage(latent, pos, rndata):
    M = latent.shape[0]
    N = pos.shape[0]
    C = rndata.shape[-1]

    cidx = jnp.clip((pos * _G).astype(jnp.int32), 0, _G - 1)
    cid = (cidx[:, 0] * _G + cidx[:, 1]) * _G + cidx[:, 2]
    order = jnp.argsort(cid).astype(jnp.int32)
    cid_sorted = cid[order]
    cs = jnp.searchsorted(cid_sorted, jnp.arange(_NCELL + 1),
                          side="left").astype(jnp.int32)

    n_sorted = ((N + 15) // 16) * 16
    csz = ((cs.shape[0] + 7) // 8) * 8
    order_p = jnp.concatenate(
        [order, jnp.full((n_sorted - N,), n_sorted - 1, jnp.int32)])
    cs_p = jnp.concatenate(
        [cs, jnp.full((csz - cs.shape[0],), N, jnp.int32)])
    big = jnp.full((n_sorted - N,), 1e6, jnp.float32)
    px = jnp.concatenate([pos[:, 0], big])
    py = jnp.concatenate([pos[:, 1], big])
    pz = jnp.concatenate([pos[:, 2], big])

    table = jnp.concatenate(
        [rndata[0], pos, jnp.zeros((N, 80 - C - 3), jnp.float32)], axis=1)
    table = jnp.concatenate([table, jnp.zeros((8, 80), jnp.float32)], axis=0)
    n_zero_row = N

    qpt = M // _NW
    init_nbr = jnp.full((qpt * _D_CAP // 128, 128), N, jnp.int32)

    mesh = plsc.VectorSubcoreMesh(core_axis_name="c", subcore_axis_name="s",
                                  num_cores=2, num_subcores=16)
    sc = pl.kernel(
        functools.partial(_sc_body, n_sorted=n_sorted, qpt=qpt,
                          n_zero_row=n_zero_row, csz=csz),
        out_type=[
            jax.ShapeDtypeStruct((M * _D_CAP, 80), jnp.float32),
            jax.ShapeDtypeStruct((M,), jnp.int32),
        ],
        mesh=mesh,
        scratch_types=[
            pltpu.VMEM((qpt,), jnp.float32),
            pltpu.VMEM((qpt,), jnp.float32),
            pltpu.VMEM((qpt,), jnp.float32),
            pltpu.VMEM((n_sorted,), jnp.float32),
            pltpu.VMEM((n_sorted,), jnp.float32),
            pltpu.VMEM((n_sorted,), jnp.float32),
            pltpu.VMEM((n_sorted,), jnp.int32),
            pltpu.VMEM((csz,), jnp.int32),
            pltpu.VMEM((qpt * _D_CAP // 128, 128), jnp.int32),
            pltpu.VMEM((qpt,), jnp.int32),
            pltpu.VMEM((128, 80), jnp.float32),
            pltpu.SemaphoreType.DMA,
        ],
        compiler_params=pltpu.CompilerParams(needs_layout_passes=False,
                                             use_tc_tiling_on_sc=False),
        interpret=_INTERPRET,
    )
    edges, counts = sc(latent[:, 0], latent[:, 1], latent[:, 2],
                       px, py, pz, order_p, cs_p, init_nbr, table)
    return edges, counts


def kernel(pos, rndata, x_coord, K0, Kb0, K1, Kb1, K2, Kb2, P0, Pb0, P1, Pb1):
    B, M, _ = x_coord.shape

    mn = x_coord.min(axis=1, keepdims=True)
    mx = x_coord.max(axis=1, keepdims=True)
    latent = ((x_coord - mn) / (mx - mn + 1e-12))[0]

    edges, counts = _sc_stage(latent, pos, rndata)
    counts_f = counts.astype(jnp.float32).reshape(M, 1)

    BQ = 128
    K0a, K0b = K0[:3], K0[3:]
    kb0 = Kb0.reshape(1, -1)
    kb1 = Kb1.reshape(1, -1)
    kb2 = Kb2.reshape(1, -1)
    pb0 = Pb0.reshape(1, -1)
    pb1 = Pb1.reshape(1, -1)

    full = lambda shp: pl.BlockSpec(shp, lambda i: tuple(0 for _ in shp))
    out = pl.pallas_call(
        functools.partial(_tc_body, bq=BQ),
        grid=(M // BQ,),
        in_specs=[
            pl.BlockSpec((BQ, 3), lambda i: (i, 0)),
            pl.BlockSpec((BQ * _D_CAP, 80), lambda i: (i, 0)),
            pl.BlockSpec((BQ, 1), lambda i: (i, 0)),
            full(K0a.shape), full(K0b.shape), full(kb0.shape),
            full(K1.shape), full(kb1.shape),
            full(K2.shape), full(kb2.shape),
            full(P0.shape), full(pb0.shape),
            full(P1.shape), full(pb1.shape),
        ],
        out_specs=pl.BlockSpec((BQ, 3), lambda i: (i, 0)),
        out_shape=jax.ShapeDtypeStruct((M, 3), jnp.float32),
        interpret=_INTERPRET,
    )(latent, edges, counts_f, K0a, K0b, kb0, K1, kb1, K2, kb2,
      P0, pb0, P1, pb1)
    return out[None]


# speedup vs baseline: 8.0430x; 8.0430x over previous
import functools

import jax
import jax.numpy as jnp
from jax import lax
from jax.experimental import pallas as pl
from jax.experimental.pallas import tpu as pltpu
from jax.experimental.pallas import tpu_sc as plsc

_RADIUS = 0.083
_R2 = _RADIUS * _RADIUS
_G = 11
_NCELL = _G * _G * _G
_RSPAN = 2
_D_CAP = 96
_LANES = 16

_NW = 32

_INTERPRET = False


def _splat_i32(x):
    return jnp.zeros((_LANES,), jnp.int32) + x


def _bf16r(v):
    b = plsc.bitcast(v, jnp.int32)
    tie = lax.bitwise_and(lax.shift_right_logical(b, 16), 1)
    b = b + 0x7FFF + tie
    b = lax.bitwise_and(b, jnp.int32(-65536))
    return plsc.bitcast(b, jnp.float32)


def _sc_body(qx_h, qy_h, qz_h, order_h, cs_h, init_h,
             pp_h, pbx_h, pby_h, pbz_h, table_h, edges_h, counts_h,
             qx_v, qy_v, qz_v, order_v, cs_v, nbr_v,
             cnt_v, gbuf_v, shared_v, pp_v, pbx_v, pby_v, pbz_v, sem,
             *, n_sorted, n_pad, qpt, n_zero_row, csz):
    sid = lax.axis_index("s")
    wid = sid * 2 + lax.axis_index("c")
    qbase = wid * qpt

    @pl.when(sid == 0)
    def _stage():
        pltpu.sync_copy(table_h, shared_v)

    pltpu.sync_copy(qx_h.at[pl.ds(qbase, qpt)], qx_v)
    pltpu.sync_copy(qy_h.at[pl.ds(qbase, qpt)], qy_v)
    pltpu.sync_copy(qz_h.at[pl.ds(qbase, qpt)], qz_v)
    pltpu.sync_copy(order_h, order_v)
    pltpu.sync_copy(cs_h, cs_v)
    pltpu.sync_copy(init_h, nbr_v)
    pltpu.sync_copy(pp_h, pp_v)
    pltpu.sync_copy(pbx_h, pbx_v)
    pltpu.sync_copy(pby_h, pby_v)
    pltpu.sync_copy(pbz_h, pbz_v)

    lane = lax.iota(jnp.int32, _LANES)
    lane0 = lane == 0

    nspan = 2 * _RSPAN + 1

    def per_query(q, _):
        qi = _splat_i32(q)
        qxv = plsc.load_gather(qx_v, [qi])
        qyv = plsc.load_gather(qy_v, [qi])
        qzv = plsc.load_gather(qz_v, [qi])
        qqv = (qxv * qxv + qyv * qyv) + qzv * qzv
        qbx = _bf16r(qxv)
        qby = _bf16r(qyv)
        qbz = _bf16r(qzv)
        cxv = jnp.clip((qxv * _G).astype(jnp.int32), 0, _G - 1)
        cyv = jnp.clip((qyv * _G).astype(jnp.int32), 0, _G - 1)
        czv = jnp.clip((qzv * _G).astype(jnp.int32), 0, _G - 1)
        zlo = jnp.maximum(czv - _RSPAN, 0)
        zhi = jnp.minimum(czv + _RSPAN, _G - 1)

        def per_run(k, cnt_vec):
            dx = k // nspan - _RSPAN
            dy = k % nspan - _RSPAN
            ax = cxv + dx
            ay = cyv + dy
            okr = (ax >= 0) & (ax < _G) & (ay >= 0) & (ay < _G)
            base = (ax * _G + ay) * _G
            lin_lo = jnp.clip(base + zlo, 0, csz - 1)
            lin_hi = jnp.clip(base + zhi + 1, 0, csz - 1)
            sv = plsc.load_gather(cs_v, [lin_lo])
            ev = plsc.load_gather(cs_v, [lin_hi])
            sv = jnp.where(okr, sv, 0)
            ev = jnp.where(okr, ev, 0)
            s_start = jnp.max(sv)
            e_end = jnp.max(ev)
            trips = (e_end - s_start + (_LANES - 1)) // _LANES

            def per_chunk(t, cnt_in):
                s0 = s_start + t * _LANES
                svec = s0 + lane
                valid = svec < e_end
                svec_c = jnp.minimum(svec, n_pad - 1)
                ov = plsc.load_gather(order_v, [svec_c])
                pp = plsc.load_gather(pp_v, [svec_c])
                pbx = plsc.load_gather(pbx_v, [svec_c])
                pby = plsc.load_gather(pby_v, [svec_c])
                pbz = plsc.load_gather(pbz_v, [svec_c])
                qp = (qbx * pbx + qby * pby) + qbz * pbz
                d2 = (qqv + pp) - 2.0 * qp
                inr = valid & (d2 <= _R2)
                pcs = plsc.cumsum(jnp.where(inr, 1, 0).astype(jnp.int32))
                tgt = cnt_in + pcs - 1
                w = inr & (tgt < _D_CAP)
                flat = jnp.clip(q * _D_CAP + tgt, 0, qpt * _D_CAP - 1)
                row = lax.shift_right_logical(flat, 7)
                col = lax.bitwise_and(flat, 127)
                plsc.store_scatter(nbr_v, [row, col], ov, mask=w)
                return cnt_in + plsc.all_reduce_population_count(inr)

            return lax.fori_loop(0, trips, per_chunk, cnt_vec)

        cnt_vec = lax.fori_loop(0, nspan * nspan, per_run, _splat_i32(0))
        plsc.store_scatter(cnt_v, [qi], cnt_vec, mask=lane0)
        return _

    lax.fori_loop(0, qpt, per_query, 0)

    pltpu.sync_copy(cnt_v, counts_h.at[pl.ds(qbase, qpt)])

    nrows = qpt * _D_CAP // 128
    ebase = qbase * _D_CAP

    plsc.subcore_barrier()

    def per_gather(c, _):
        pltpu.async_copy(shared_v.at[nbr_v.at[c]], gbuf_v, sem).wait()
        pltpu.sync_copy(gbuf_v, edges_h.at[pl.ds(ebase + c * 128, 128)])
        return _

    lax.fori_loop(0, nrows, per_gather, 0)


def _tc_body(lat_ref, edges_ref, cnt_ref,
             k0a_ref, k0b_ref, kb0_ref, k1_ref, kb1_ref, k2_ref, kb2_ref,
             p0_ref, pb0_ref, p1_ref, pb1_ref, out_ref, *, bq):
    e = edges_ref[...]
    rb = e[:, :64]
    pe = e[:, 64:67]
    lat = lat_ref[...]
    aq = jnp.dot(lat, k0a_ref[...]) + kb0_ref[...]
    aqe = jnp.broadcast_to(aq[:, None, :], (bq, _D_CAP, 64))
    aqe = aqe.reshape(bq * _D_CAP, 64)
    h1 = jax.nn.gelu(aqe + jnp.dot(pe, k0b_ref[...]))
    h2 = jax.nn.gelu(jnp.dot(h1, k1_ref[...]) + kb1_ref[...])
    kv = jnp.dot(h2, k2_ref[...]) + kb2_ref[...]
    v = kv * rb
    s = v.reshape(bq, _D_CAP, 64).sum(axis=1)
    cnt = jnp.clip(cnt_ref[...], 1.0, None)
    mean = s / cnt
    h = jax.nn.gelu(jnp.dot(mean, p0_ref[...]) + pb0_ref[...])
    out_ref[...] = jnp.dot(h, p1_ref[...]) + pb1_ref[...]


def _sc_stage(latent, pos, rndata):
    M = latent.shape[0]
    N = pos.shape[0]
    C = rndata.shape[-1]

    cidx = jnp.clip((pos * _G).astype(jnp.int32), 0, _G - 1)
    cid = (cidx[:, 0] * _G + cidx[:, 1]) * _G + cidx[:, 2]
    order = jnp.argsort(cid).astype(jnp.int32)
    cid_sorted = cid[order]
    cs = jnp.searchsorted(cid_sorted, jnp.arange(_NCELL + 1),
                          side="left").astype(jnp.int32)

    n_sorted = ((N + 15) // 16) * 16
    n_pad = n_sorted + 16
    csz = ((cs.shape[0] + 7) // 8) * 8
    order_p = jnp.concatenate(
        [order, jnp.full((n_pad - N,), N, jnp.int32)])
    cs_p = jnp.concatenate(
        [cs, jnp.full((csz - cs.shape[0],), N, jnp.int32)])
    pos_s = jnp.concatenate(
        [pos[order], jnp.full((n_pad - N, 3), 1e6, jnp.float32)], axis=0)
    pp_s = (pos_s[:, 0] * pos_s[:, 0] + pos_s[:, 1] * pos_s[:, 1]) \
        + pos_s[:, 2] * pos_s[:, 2]
    pb = lax.bitcast_convert_type(pos_s, jnp.int32)
    pb = pb + 0x7FFF + lax.bitwise_and(lax.shift_right_logical(pb, 16), 1)
    pos_b = lax.bitcast_convert_type(
        lax.bitwise_and(pb, jnp.int32(-65536)), jnp.float32)

    table = jnp.concatenate(
        [rndata[0], pos, jnp.zeros((N, 80 - C - 3), jnp.float32)], axis=1)
    table = jnp.concatenate([table, jnp.zeros((8, 80), jnp.float32)], axis=0)
    n_zero_row = N

    qpt = M // _NW
    init_nbr = jnp.full((qpt * _D_CAP // 128, 128), N, jnp.int32)

    mesh = plsc.VectorSubcoreMesh(core_axis_name="c", subcore_axis_name="s",
                                  num_cores=2, num_subcores=16)
    sc = pl.kernel(
        functools.partial(_sc_body, n_sorted=n_sorted, n_pad=n_pad, qpt=qpt,
                          n_zero_row=n_zero_row, csz=csz),
        out_type=[
            jax.ShapeDtypeStruct((M * _D_CAP, 80), jnp.float32),
            jax.ShapeDtypeStruct((M,), jnp.int32),
        ],
        mesh=mesh,
        scratch_types=[
            pltpu.VMEM((qpt,), jnp.float32),
            pltpu.VMEM((qpt,), jnp.float32),
            pltpu.VMEM((qpt,), jnp.float32),
            pltpu.VMEM((n_pad,), jnp.int32),
            pltpu.VMEM((csz,), jnp.int32),
            pltpu.VMEM((qpt * _D_CAP // 128, 128), jnp.int32),
            pltpu.VMEM((qpt,), jnp.int32),
            pltpu.VMEM((128, 80), jnp.float32),
            pltpu.VMEM_SHARED((N + 8, 80), jnp.float32),
            pltpu.VMEM((n_pad,), jnp.float32),
            pltpu.VMEM((n_pad,), jnp.float32),
            pltpu.VMEM((n_pad,), jnp.float32),
            pltpu.VMEM((n_pad,), jnp.float32),
            pltpu.SemaphoreType.DMA,
        ],
        compiler_params=pltpu.CompilerParams(needs_layout_passes=False,
                                             use_tc_tiling_on_sc=False),
        interpret=_INTERPRET,
    )
    edges, counts = sc(latent[:, 0], latent[:, 1], latent[:, 2],
                       order_p, cs_p, init_nbr,
                       pp_s, pos_b[:, 0], pos_b[:, 1], pos_b[:, 2], table)
    return edges, counts


def kernel(pos, rndata, x_coord, K0, Kb0, K1, Kb1, K2, Kb2, P0, Pb0, P1, Pb1):
    B, M, _ = x_coord.shape

    mn = x_coord.min(axis=1, keepdims=True)
    mx = x_coord.max(axis=1, keepdims=True)
    latent = ((x_coord - mn) / (mx - mn + 1e-12))[0]

    edges, counts = _sc_stage(latent, pos, rndata)
    counts_f = counts.astype(jnp.float32).reshape(M, 1)

    BQ = 128
    K0a, K0b = K0[:3], K0[3:]
    kb0 = Kb0.reshape(1, -1)
    kb1 = Kb1.reshape(1, -1)
    kb2 = Kb2.reshape(1, -1)
    pb0 = Pb0.reshape(1, -1)
    pb1 = Pb1.reshape(1, -1)

    full = lambda shp: pl.BlockSpec(shp, lambda i: tuple(0 for _ in shp))
    out = pl.pallas_call(
        functools.partial(_tc_body, bq=BQ),
        grid=(M // BQ,),
        in_specs=[
            pl.BlockSpec((BQ, 3), lambda i: (i, 0)),
            pl.BlockSpec((BQ * _D_CAP, 80), lambda i: (i, 0)),
            pl.BlockSpec((BQ, 1), lambda i: (i, 0)),
            full(K0a.shape), full(K0b.shape), full(kb0.shape),
            full(K1.shape), full(kb1.shape),
            full(K2.shape), full(kb2.shape),
            full(P0.shape), full(pb0.shape),
            full(P1.shape), full(pb1.shape),
        ],
        out_specs=pl.BlockSpec((BQ, 3), lambda i: (i, 0)),
        out_shape=jax.ShapeDtypeStruct((M, 3), jnp.float32),
        interpret=_INTERPRET,
    )(latent, edges, counts_f, K0a, K0b, kb0, K1, kb1, K2, kb2,
      P0, pb0, P1, pb1)
    return out[None]


# speedup vs baseline: 8.5134x; 1.0585x over previous
import functools

import jax
import jax.numpy as jnp
from jax import lax
from jax.experimental import pallas as pl
from jax.experimental.pallas import tpu as pltpu
from jax.experimental.pallas import tpu_sc as plsc

_RADIUS = 0.083
_R2 = _RADIUS * _RADIUS
_G = 11
_NCELL = _G * _G * _G
_RSPAN = 2
_D_CAP = 96
_LANES = 16

_NW = 32

_INTERPRET = False


def _splat_i32(x):
    return jnp.zeros((_LANES,), jnp.int32) + x


def _bf16r(v):
    b = plsc.bitcast(v, jnp.int32)
    tie = lax.bitwise_and(lax.shift_right_logical(b, 16), 1)
    b = b + 0x7FFF + tie
    b = lax.bitwise_and(b, jnp.int32(-65536))
    return plsc.bitcast(b, jnp.float32)


def _sc_body(qx_h, qy_h, qz_h, order_h, cs_h, init_h,
             pp_h, pbx_h, pby_h, pbz_h, table_h, edges_h, counts_h,
             qx_v, qy_v, qz_v, order_v, cs_v, nbr_v,
             cnt_v, gbuf_v, shared_v, pp_v, pbx_v, pby_v, pbz_v,
             runs_v, rune_v, sem,
             *, n_sorted, n_pad, qpt, n_zero_row, csz):
    sid = lax.axis_index("s")
    wid = sid * 2 + lax.axis_index("c")
    qbase = wid * qpt

    @pl.when(sid == 0)
    def _stage():
        pltpu.sync_copy(table_h, shared_v)

    pltpu.sync_copy(qx_h.at[pl.ds(qbase, qpt)], qx_v)
    pltpu.sync_copy(qy_h.at[pl.ds(qbase, qpt)], qy_v)
    pltpu.sync_copy(qz_h.at[pl.ds(qbase, qpt)], qz_v)
    pltpu.sync_copy(order_h, order_v)
    pltpu.sync_copy(cs_h, cs_v)
    pltpu.sync_copy(init_h, nbr_v)
    pltpu.sync_copy(pp_h, pp_v)
    pltpu.sync_copy(pbx_h, pbx_v)
    pltpu.sync_copy(pby_h, pby_v)
    pltpu.sync_copy(pbz_h, pbz_v)

    lane = lax.iota(jnp.int32, _LANES)
    lane0 = lane == 0

    nspan = 2 * _RSPAN + 1

    def per_query(q, _):
        qi = _splat_i32(q)
        qxv = plsc.load_gather(qx_v, [qi])
        qyv = plsc.load_gather(qy_v, [qi])
        qzv = plsc.load_gather(qz_v, [qi])
        qqv = (qxv * qxv + qyv * qyv) + qzv * qzv
        qbx = _bf16r(qxv)
        qby = _bf16r(qyv)
        qbz = _bf16r(qzv)
        cxv = jnp.clip((qxv * _G).astype(jnp.int32), 0, _G - 1)
        cyv = jnp.clip((qyv * _G).astype(jnp.int32), 0, _G - 1)
        czv = jnp.clip((qzv * _G).astype(jnp.int32), 0, _G - 1)
        zlo = jnp.maximum(czv - _RSPAN, 0)
        zhi = jnp.minimum(czv + _RSPAN, _G - 1)

        for h in range(2):
            kk = lane + h * _LANES
            dxv = kk // nspan - _RSPAN
            dyv = kk % nspan - _RSPAN
            axv = cxv + dxv
            ayv = cyv + dyv
            okv = ((axv >= 0) & (axv < _G) & (ayv >= 0) & (ayv < _G)
                   & (kk < nspan * nspan))
            basev = (axv * _G + ayv) * _G
            lo = jnp.clip(basev + zlo, 0, csz - 1)
            hi = jnp.clip(basev + zhi + 1, 0, csz - 1)
            sv = jnp.where(okv, plsc.load_gather(cs_v, [lo]), 0)
            ev = jnp.where(okv, plsc.load_gather(cs_v, [hi]), 0)
            runs_v[pl.ds(h * _LANES, _LANES)] = sv
            rune_v[pl.ds(h * _LANES, _LANES)] = ev
        runs_v[pl.ds(2 * _LANES, _LANES)] = jnp.zeros((_LANES,), jnp.int32)
        rune_v[pl.ds(2 * _LANES, _LANES)] = jnp.zeros((_LANES,), jnp.int32)

        def per_run(k, cnt_vec):
            s_start = runs_v[pl.ds(k, _LANES)][0]
            e_end = rune_v[pl.ds(k, _LANES)][0]
            trips = (e_end - s_start + (_LANES - 1)) // _LANES

            def per_chunk(t, cnt_in):
                s0 = s_start + t * _LANES
                svec = s0 + lane
                valid = svec < e_end
                svec_c = jnp.minimum(svec, n_pad - 1)
                ov = plsc.load_gather(order_v, [svec_c])
                pp = plsc.load_gather(pp_v, [svec_c])
                pbx = plsc.load_gather(pbx_v, [svec_c])
                pby = plsc.load_gather(pby_v, [svec_c])
                pbz = plsc.load_gather(pbz_v, [svec_c])
                qp = (qbx * pbx + qby * pby) + qbz * pbz
                d2 = (qqv + pp) - 2.0 * qp
                inr = valid & (d2 <= _R2)
                pcs = plsc.cumsum(jnp.where(inr, 1, 0).astype(jnp.int32))
                tgt = cnt_in + pcs - 1
                w = inr & (tgt < _D_CAP)
                flat = jnp.clip(q * _D_CAP + tgt, 0, qpt * _D_CAP - 1)
                row = lax.shift_right_logical(flat, 7)
                col = lax.bitwise_and(flat, 127)
                plsc.store_scatter(nbr_v, [row, col], ov, mask=w)
                return cnt_in + plsc.all_reduce_population_count(inr)

            return lax.fori_loop(0, trips, per_chunk, cnt_vec)

        cnt_vec = lax.fori_loop(0, nspan * nspan, per_run, _splat_i32(0))
        plsc.store_scatter(cnt_v, [qi], cnt_vec, mask=lane0)
        return _

    lax.fori_loop(0, qpt, per_query, 0)

    pltpu.sync_copy(cnt_v, counts_h.at[pl.ds(qbase, qpt)])

    nrows = qpt * _D_CAP // 128
    ebase = qbase * _D_CAP

    plsc.subcore_barrier()

    def per_gather(c, _):
        pltpu.async_copy(shared_v.at[nbr_v.at[c]], gbuf_v, sem).wait()
        pltpu.sync_copy(gbuf_v, edges_h.at[pl.ds(ebase + c * 128, 128)])
        return _

    lax.fori_loop(0, nrows, per_gather, 0)


def _tc_body(lat_ref, edges_ref, cnt_ref,
             k0a_ref, k0b_ref, kb0_ref, k1_ref, kb1_ref, k2_ref, kb2_ref,
             p0_ref, pb0_ref, p1_ref, pb1_ref, out_ref, *, bq):
    e = edges_ref[...]
    rb = e[:, :64]
    pe = e[:, 64:67]
    lat = lat_ref[...]
    aq = jnp.dot(lat, k0a_ref[...]) + kb0_ref[...]
    aqe = jnp.broadcast_to(aq[:, None, :], (bq, _D_CAP, 64))
    aqe = aqe.reshape(bq * _D_CAP, 64)
    h1 = jax.nn.gelu(aqe + jnp.dot(pe, k0b_ref[...]))
    h2 = jax.nn.gelu(jnp.dot(h1, k1_ref[...]) + kb1_ref[...])
    kv = jnp.dot(h2, k2_ref[...]) + kb2_ref[...]
    v = kv * rb
    s = v.reshape(bq, _D_CAP, 64).sum(axis=1)
    cnt = jnp.clip(cnt_ref[...], 1.0, None)
    mean = s / cnt
    h = jax.nn.gelu(jnp.dot(mean, p0_ref[...]) + pb0_ref[...])
    out_ref[...] = jnp.dot(h, p1_ref[...]) + pb1_ref[...]


def _sc_stage(latent, pos, rndata):
    M = latent.shape[0]
    N = pos.shape[0]
    C = rndata.shape[-1]

    cidx = jnp.clip((pos * _G).astype(jnp.int32), 0, _G - 1)
    cid = (cidx[:, 0] * _G + cidx[:, 1]) * _G + cidx[:, 2]
    order = jnp.argsort(cid).astype(jnp.int32)
    cid_sorted = cid[order]
    cs = jnp.searchsorted(cid_sorted, jnp.arange(_NCELL + 1),
                          side="left").astype(jnp.int32)

    n_sorted = ((N + 15) // 16) * 16
    n_pad = n_sorted + 16
    csz = ((cs.shape[0] + 7) // 8) * 8
    order_p = jnp.concatenate(
        [order, jnp.full((n_pad - N,), N, jnp.int32)])
    cs_p = jnp.concatenate(
        [cs, jnp.full((csz - cs.shape[0],), N, jnp.int32)])
    pos_s = jnp.concatenate(
        [pos[order], jnp.full((n_pad - N, 3), 1e6, jnp.float32)], axis=0)
    pp_s = (pos_s[:, 0] * pos_s[:, 0] + pos_s[:, 1] * pos_s[:, 1]) \
        + pos_s[:, 2] * pos_s[:, 2]
    pb = lax.bitcast_convert_type(pos_s, jnp.int32)
    pb = pb + 0x7FFF + lax.bitwise_and(lax.shift_right_logical(pb, 16), 1)
    pos_b = lax.bitcast_convert_type(
        lax.bitwise_and(pb, jnp.int32(-65536)), jnp.float32)

    table = jnp.concatenate(
        [rndata[0], pos, jnp.zeros((N, 80 - C - 3), jnp.float32)], axis=1)
    table = jnp.concatenate([table, jnp.zeros((8, 80), jnp.float32)], axis=0)
    n_zero_row = N

    qpt = M // _NW
    init_nbr = jnp.full((qpt * _D_CAP // 128, 128), N, jnp.int32)

    mesh = plsc.VectorSubcoreMesh(core_axis_name="c", subcore_axis_name="s",
                                  num_cores=2, num_subcores=16)
    sc = pl.kernel(
        functools.partial(_sc_body, n_sorted=n_sorted, n_pad=n_pad, qpt=qpt,
                          n_zero_row=n_zero_row, csz=csz),
        out_type=[
            jax.ShapeDtypeStruct((M * _D_CAP, 80), jnp.float32),
            jax.ShapeDtypeStruct((M,), jnp.int32),
        ],
        mesh=mesh,
        scratch_types=[
            pltpu.VMEM((qpt,), jnp.float32),
            pltpu.VMEM((qpt,), jnp.float32),
            pltpu.VMEM((qpt,), jnp.float32),
            pltpu.VMEM((n_pad,), jnp.int32),
            pltpu.VMEM((csz,), jnp.int32),
            pltpu.VMEM((qpt * _D_CAP // 128, 128), jnp.int32),
            pltpu.VMEM((qpt,), jnp.int32),
            pltpu.VMEM((128, 80), jnp.float32),
            pltpu.VMEM_SHARED((N + 8, 80), jnp.float32),
            pltpu.VMEM((n_pad,), jnp.float32),
            pltpu.VMEM((n_pad,), jnp.float32),
            pltpu.VMEM((n_pad,), jnp.float32),
            pltpu.VMEM((n_pad,), jnp.float32),
            pltpu.VMEM((3 * _LANES,), jnp.int32),
            pltpu.VMEM((3 * _LANES,), jnp.int32),
            pltpu.SemaphoreType.DMA,
        ],
        compiler_params=pltpu.CompilerParams(needs_layout_passes=False,
                                             use_tc_tiling_on_sc=False),
        interpret=_INTERPRET,
    )
    edges, counts = sc(latent[:, 0], latent[:, 1], latent[:, 2],
                       order_p, cs_p, init_nbr,
                       pp_s, pos_b[:, 0], pos_b[:, 1], pos_b[:, 2], table)
    return edges, counts


def kernel(pos, rndata, x_coord, K0, Kb0, K1, Kb1, K2, Kb2, P0, Pb0, P1, Pb1):
    B, M, _ = x_coord.shape

    mn = x_coord.min(axis=1, keepdims=True)
    mx = x_coord.max(axis=1, keepdims=True)
    latent = ((x_coord - mn) / (mx - mn + 1e-12))[0]

    edges, counts = _sc_stage(latent, pos, rndata)
    counts_f = counts.astype(jnp.float32).reshape(M, 1)

    BQ = 128
    K0a, K0b = K0[:3], K0[3:]
    kb0 = Kb0.reshape(1, -1)
    kb1 = Kb1.reshape(1, -1)
    kb2 = Kb2.reshape(1, -1)
    pb0 = Pb0.reshape(1, -1)
    pb1 = Pb1.reshape(1, -1)

    full = lambda shp: pl.BlockSpec(shp, lambda i: tuple(0 for _ in shp))
    out = pl.pallas_call(
        functools.partial(_tc_body, bq=BQ),
        grid=(M // BQ,),
        in_specs=[
            pl.BlockSpec((BQ, 3), lambda i: (i, 0)),
            pl.BlockSpec((BQ * _D_CAP, 80), lambda i: (i, 0)),
            pl.BlockSpec((BQ, 1), lambda i: (i, 0)),
            full(K0a.shape), full(K0b.shape), full(kb0.shape),
            full(K1.shape), full(kb1.shape),
            full(K2.shape), full(kb2.shape),
            full(P0.shape), full(pb0.shape),
            full(P1.shape), full(pb1.shape),
        ],
        out_specs=pl.BlockSpec((BQ, 3), lambda i: (i, 0)),
        out_shape=jax.ShapeDtypeStruct((M, 3), jnp.float32),
        interpret=_INTERPRET,
    )(latent, edges, counts_f, K0a, K0b, kb0, K1, kb1, K2, kb2,
      P0, pb0, P1, pb1)
    return out[None]


# speedup vs baseline: 8.5177x; 1.0005x over previous
import functools

import jax
import jax.numpy as jnp
from jax import lax
from jax.experimental import pallas as pl
from jax.experimental.pallas import tpu as pltpu
from jax.experimental.pallas import tpu_sc as plsc

_RADIUS = 0.083
_R2 = _RADIUS * _RADIUS
_G = 11
_NCELL = _G * _G * _G
_RSPAN = 2
_D_CAP = 96
_LANES = 16

_NW = 32


def _splat_i32(x):
    return jnp.zeros((_LANES,), jnp.int32) + x


def _bf16r(v):
    b = plsc.bitcast(v, jnp.int32)
    tie = lax.bitwise_and(lax.shift_right_logical(b, 16), 1)
    b = b + 0x7FFF + tie
    b = lax.bitwise_and(b, jnp.int32(-65536))
    return plsc.bitcast(b, jnp.float32)


def _sc_body(qx_h, qy_h, qz_h, order_h, cs_h, init_h,
             pp_h, pbx_h, pby_h, pbz_h, table_h, edges_h, counts_h,
             qx_v, qy_v, qz_v, order_v, cs_v, nbr_v,
             cnt_v, gbuf_v, shared_v, pp_v, pbx_v, pby_v, pbz_v,
             runs_v, rune_v, sem,
             *, n_sorted, n_pad, qpt, n_zero_row, csz):
    sid = lax.axis_index("s")
    wid = sid * 2 + lax.axis_index("c")
    qbase = wid * qpt

    @pl.when(sid == 0)
    def _stage():
        pltpu.sync_copy(table_h, shared_v)

    pltpu.sync_copy(qx_h.at[pl.ds(qbase, qpt)], qx_v)
    pltpu.sync_copy(qy_h.at[pl.ds(qbase, qpt)], qy_v)
    pltpu.sync_copy(qz_h.at[pl.ds(qbase, qpt)], qz_v)
    pltpu.sync_copy(order_h, order_v)
    pltpu.sync_copy(cs_h, cs_v)
    pltpu.sync_copy(init_h, nbr_v)
    pltpu.sync_copy(pp_h, pp_v)
    pltpu.sync_copy(pbx_h, pbx_v)
    pltpu.sync_copy(pby_h, pby_v)
    pltpu.sync_copy(pbz_h, pbz_v)

    lane = lax.iota(jnp.int32, _LANES)
    lane0 = lane == 0

    nspan = 2 * _RSPAN + 1

    def per_query(q, _):
        qi = _splat_i32(q)
        qxv = plsc.load_gather(qx_v, [qi])
        qyv = plsc.load_gather(qy_v, [qi])
        qzv = plsc.load_gather(qz_v, [qi])
        qqv = (qxv * qxv + qyv * qyv) + qzv * qzv
        qbx = _bf16r(qxv)
        qby = _bf16r(qyv)
        qbz = _bf16r(qzv)
        cxv = jnp.clip((qxv * _G).astype(jnp.int32), 0, _G - 1)
        cyv = jnp.clip((qyv * _G).astype(jnp.int32), 0, _G - 1)
        czv = jnp.clip((qzv * _G).astype(jnp.int32), 0, _G - 1)
        zlo = jnp.maximum(czv - _RSPAN, 0)
        zhi = jnp.minimum(czv + _RSPAN, _G - 1)

        for h in range(2):
            kk = lane + h * _LANES
            dxv = kk // nspan - _RSPAN
            dyv = kk % nspan - _RSPAN
            axv = cxv + dxv
            ayv = cyv + dyv
            okv = ((axv >= 0) & (axv < _G) & (ayv >= 0) & (ayv < _G)
                   & (kk < nspan * nspan))
            basev = (axv * _G + ayv) * _G
            lo = jnp.clip(basev + zlo, 0, csz - 1)
            hi = jnp.clip(basev + zhi + 1, 0, csz - 1)
            sv = jnp.where(okv, plsc.load_gather(cs_v, [lo]), 0)
            ev = jnp.where(okv, plsc.load_gather(cs_v, [hi]), 0)
            runs_v[pl.ds(h * _LANES, _LANES)] = sv
            rune_v[pl.ds(h * _LANES, _LANES)] = ev
        runs_v[pl.ds(2 * _LANES, _LANES)] = jnp.zeros((_LANES,), jnp.int32)
        rune_v[pl.ds(2 * _LANES, _LANES)] = jnp.zeros((_LANES,), jnp.int32)

        def per_run(k, cnt_vec):
            s_start = runs_v[pl.ds(k, _LANES)][0]
            e_end = rune_v[pl.ds(k, _LANES)][0]
            trips = (e_end - s_start + (_LANES - 1)) // _LANES

            def per_chunk(t, cnt_in):
                s0 = s_start + t * _LANES
                svec = s0 + lane
                valid = svec < e_end
                svec_c = jnp.minimum(svec, n_pad - 1)
                ov = plsc.load_gather(order_v, [svec_c])
                pp = plsc.load_gather(pp_v, [svec_c])
                pbx = plsc.load_gather(pbx_v, [svec_c])
                pby = plsc.load_gather(pby_v, [svec_c])
                pbz = plsc.load_gather(pbz_v, [svec_c])
                qp = (qbx * pbx + qby * pby) + qbz * pbz
                d2 = (qqv + pp) - 2.0 * qp
                inr = valid & (d2 <= _R2)
                pcs = plsc.cumsum(jnp.where(inr, 1, 0).astype(jnp.int32))
                tgt = cnt_in + pcs - 1
                w = inr & (tgt < _D_CAP)
                flat = jnp.clip(q * _D_CAP + tgt, 0, qpt * _D_CAP - 1)
                row = lax.shift_right_logical(flat, 7)
                col = lax.bitwise_and(flat, 127)
                plsc.store_scatter(nbr_v, [row, col], ov, mask=w)
                return cnt_in + plsc.all_reduce_population_count(inr)

            return lax.fori_loop(0, trips, per_chunk, cnt_vec)

        cnt_vec = lax.fori_loop(0, nspan * nspan, per_run, _splat_i32(0))
        plsc.store_scatter(cnt_v, [qi], cnt_vec, mask=lane0)
        return _

    lax.fori_loop(0, qpt, per_query, 0)

    pltpu.sync_copy(cnt_v, counts_h.at[pl.ds(qbase, qpt)])

    nrows = qpt * _D_CAP // 128
    ebase = qbase * _D_CAP

    plsc.subcore_barrier()

    def per_gather(c, _):
        pltpu.async_copy(shared_v.at[nbr_v.at[c]], gbuf_v, sem).wait()
        pltpu.sync_copy(gbuf_v, edges_h.at[pl.ds(ebase + c * 128, 128)])
        return _

    lax.fori_loop(0, nrows, per_gather, 0)


def _tc_body(lat_ref, edges_ref, cnt_ref,
             k0a_ref, k0b_ref, kb0_ref, k1_ref, kb1_ref, k2_ref, kb2_ref,
             p0_ref, pb0_ref, p1_ref, pb1_ref, out_ref, *, bq):
    e = edges_ref[...]
    rb = e[:, :64]
    pe = e[:, 64:67]
    lat = lat_ref[...]
    aq = jnp.dot(lat, k0a_ref[...]) + kb0_ref[...]
    aqe = jnp.broadcast_to(aq[:, None, :], (bq, _D_CAP, 64))
    aqe = aqe.reshape(bq * _D_CAP, 64)
    h1 = jax.nn.gelu(aqe + jnp.dot(pe, k0b_ref[...]))
    h2 = jax.nn.gelu(jnp.dot(h1, k1_ref[...]) + kb1_ref[...])
    kv = jnp.dot(h2, k2_ref[...]) + kb2_ref[...]
    v = kv * rb
    s = v.reshape(bq, _D_CAP, 64).sum(axis=1)
    cnt = jnp.clip(cnt_ref[...], 1.0, None)
    mean = s / cnt
    h = jax.nn.gelu(jnp.dot(mean, p0_ref[...]) + pb0_ref[...])
    out_ref[...] = jnp.dot(h, p1_ref[...]) + pb1_ref[...]


def _sc_stage(latent, pos, rndata):
    M = latent.shape[0]
    N = pos.shape[0]
    C = rndata.shape[-1]

    cidx = jnp.clip((pos * _G).astype(jnp.int32), 0, _G - 1)
    cid = (cidx[:, 0] * _G + cidx[:, 1]) * _G + cidx[:, 2]
    order = jnp.argsort(cid).astype(jnp.int32)
    cid_sorted = cid[order]
    cs = jnp.searchsorted(cid_sorted, jnp.arange(_NCELL + 1),
                          side="left").astype(jnp.int32)

    n_sorted = ((N + 15) // 16) * 16
    n_pad = n_sorted + 16
    csz = ((cs.shape[0] + 7) // 8) * 8
    order_p = jnp.concatenate(
        [order, jnp.full((n_pad - N,), N, jnp.int32)])
    cs_p = jnp.concatenate(
        [cs, jnp.full((csz - cs.shape[0],), N, jnp.int32)])
    pos_s = jnp.concatenate(
        [pos[order], jnp.full((n_pad - N, 3), 1e6, jnp.float32)], axis=0)
    pp_s = (pos_s[:, 0] * pos_s[:, 0] + pos_s[:, 1] * pos_s[:, 1]) \
        + pos_s[:, 2] * pos_s[:, 2]
    pb = lax.bitcast_convert_type(pos_s, jnp.int32)
    pb = pb + 0x7FFF + lax.bitwise_and(lax.shift_right_logical(pb, 16), 1)
    pos_b = lax.bitcast_convert_type(
        lax.bitwise_and(pb, jnp.int32(-65536)), jnp.float32)

    table = jnp.concatenate(
        [rndata[0], pos, jnp.zeros((N, 80 - C - 3), jnp.float32)], axis=1)
    table = jnp.concatenate([table, jnp.zeros((8, 80), jnp.float32)], axis=0)
    n_zero_row = N

    qpt = M // _NW
    init_nbr = jnp.full((qpt * _D_CAP // 128, 128), N, jnp.int32)

    mesh = plsc.VectorSubcoreMesh(core_axis_name="c", subcore_axis_name="s",
                                  num_cores=2, num_subcores=16)
    sc = pl.kernel(
        functools.partial(_sc_body, n_sorted=n_sorted, n_pad=n_pad, qpt=qpt,
                          n_zero_row=n_zero_row, csz=csz),
        out_type=[
            jax.ShapeDtypeStruct((M * _D_CAP, 80), jnp.float32),
            jax.ShapeDtypeStruct((M,), jnp.int32),
        ],
        mesh=mesh,
        scratch_types=[
            pltpu.VMEM((qpt,), jnp.float32),
            pltpu.VMEM((qpt,), jnp.float32),
            pltpu.VMEM((qpt,), jnp.float32),
            pltpu.VMEM((n_pad,), jnp.int32),
            pltpu.VMEM((csz,), jnp.int32),
            pltpu.VMEM((qpt * _D_CAP // 128, 128), jnp.int32),
            pltpu.VMEM((qpt,), jnp.int32),
            pltpu.VMEM((128, 80), jnp.float32),
            pltpu.VMEM_SHARED((N + 8, 80), jnp.float32),
            pltpu.VMEM((n_pad,), jnp.float32),
            pltpu.VMEM((n_pad,), jnp.float32),
            pltpu.VMEM((n_pad,), jnp.float32),
            pltpu.VMEM((n_pad,), jnp.float32),
            pltpu.VMEM((3 * _LANES,), jnp.int32),
            pltpu.VMEM((3 * _LANES,), jnp.int32),
            pltpu.SemaphoreType.DMA,
        ],
        compiler_params=pltpu.CompilerParams(needs_layout_passes=False,
                                             use_tc_tiling_on_sc=False),
    )
    edges, counts = sc(latent[:, 0], latent[:, 1], latent[:, 2],
                       order_p, cs_p, init_nbr,
                       pp_s, pos_b[:, 0], pos_b[:, 1], pos_b[:, 2], table)
    return edges, counts


def kernel(pos, rndata, x_coord, K0, Kb0, K1, Kb1, K2, Kb2, P0, Pb0, P1, Pb1):
    B, M, _ = x_coord.shape

    mn = x_coord.min(axis=1, keepdims=True)
    mx = x_coord.max(axis=1, keepdims=True)
    latent = ((x_coord - mn) / (mx - mn + 1e-12))[0]

    edges, counts = _sc_stage(latent, pos, rndata)
    counts_f = counts.astype(jnp.float32).reshape(M, 1)

    BQ = 128
    K0a, K0b = K0[:3], K0[3:]
    kb0 = Kb0.reshape(1, -1)
    kb1 = Kb1.reshape(1, -1)
    kb2 = Kb2.reshape(1, -1)
    pb0 = Pb0.reshape(1, -1)
    pb1 = Pb1.reshape(1, -1)

    full = lambda shp: pl.BlockSpec(shp, lambda i: tuple(0 for _ in shp))
    out = pl.pallas_call(
        functools.partial(_tc_body, bq=BQ),
        grid=(M // BQ,),
        in_specs=[
            pl.BlockSpec((BQ, 3), lambda i: (i, 0)),
            pl.BlockSpec((BQ * _D_CAP, 80), lambda i: (i, 0)),
            pl.BlockSpec((BQ, 1), lambda i: (i, 0)),
            full(K0a.shape), full(K0b.shape), full(kb0.shape),
            full(K1.shape), full(kb1.shape),
            full(K2.shape), full(kb2.shape),
            full(P0.shape), full(pb0.shape),
            full(P1.shape), full(pb1.shape),
        ],
        out_specs=pl.BlockSpec((BQ, 3), lambda i: (i, 0)),
        out_shape=jax.ShapeDtypeStruct((M, 3), jnp.float32),
    )(latent, edges, counts_f, K0a, K0b, kb0, K1, kb1, K2, kb2,
      P0, pb0, P1, pb1)
    return out[None]
